# baseline (speedup 1.0000x reference)
import functools

import jax
import jax.numpy as jnp
from jax import lax
from jax.experimental import pallas as pl
from jax.experimental.pallas import tpu as pltpu
from jax.experimental.pallas import tpu_sc as plsc

NC = 2
NS = 16
DH = 128
CH = 64
NSLOT = 162
SBLK = 54


def _pre_body(x_ref, w_ref, out_ref):
    y = lax.dot_general(x_ref[...], w_ref[...], (((1,), (1,)), ((), ())),
                        preferred_element_type=jnp.float32)
    y = jnp.maximum(y, 0.0)
    out_ref[0] = y[:, :DH]
    out_ref[1] = y[:, DH:]


def _pre_matmul(h_neighbour, w_pre, bn, n_pad):
    n, d = h_neighbour.shape
    grid = n // bn
    out = pl.pallas_call(
        _pre_body,
        grid=(grid,),
        in_specs=[
            pl.BlockSpec((bn, d), lambda i: (i, 0)),
            pl.BlockSpec((d, d), lambda i: (0, 0)),
        ],
        out_specs=pl.BlockSpec((NC, bn, DH), lambda i: (0, i, 0)),
        out_shape=jax.ShapeDtypeStruct((NC, n_pad, DH), jnp.float32),
    )(h_neighbour, w_pre)
    return out.reshape(NC * n_pad, DH)


def _sum_body(n_pad, e, table_ref, src_ref, dst_ref, out_s_ref,
              src_blk, d_a, d_b, d_c, rows_a, rows_b, rows_c, acc,
              sg_a, sg_b, sg_c, ss_a, ss_b, ss_c, si_a, si_b, si_c):
    c = lax.axis_index("c")
    s = lax.axis_index("s")
    rpt = n_pad // NS
    base = s * rpt
    spt = NSLOT * CH

    zero16 = jnp.zeros((16,), jnp.float32)

    def zero_rows(i, _):
        rows_a[i // (DH // 16), pl.ds((i % (DH // 16)) * 16, 16)] = zero16
        return 0
    lax.fori_loop(0, CH * (DH // 16), zero_rows, 0)

    nzc = rpt // CH
    rem = rpt % CH

    def zero_acc(i, _):
        pltpu.sync_copy(rows_a, acc.at[pl.ds(base + i * CH, CH)])
        return 0
    lax.fori_loop(0, nzc, zero_acc, 0)
    if rem:
        pltpu.sync_copy(rows_a.at[pl.ds(0, rem)], acc.at[pl.ds(base + nzc * CH, rem)])

    plsc.subcore_barrier()

    sbase = (c * NS + s) * spt
    dbase = s * spt

    nsb = NSLOT // SBLK

    def superblock(sb, _):
        soff = sb * SBLK * CH
        pltpu.sync_copy(src_ref.at[pl.ds(sbase + soff, SBLK * CH)], src_blk)

        def gath(j, rows, sem):
            pltpu.async_copy(table_ref.at[src_blk.at[pl.ds(j * CH, CH)]],
                             rows, sem)

        def dcpy(j, d_v, sem):
            pltpu.async_copy(dst_ref.at[pl.ds(dbase + soff + j * CH, CH)],
                             d_v, sem)

        def scat(rows, d_v, sem):
            pltpu.async_copy(rows, acc.at[d_v], sem, add=True)

        def wait_g(rows, sem):
            pltpu.make_async_copy(table_ref.at[pl.ds(0, CH)], rows, sem).wait()

        def wait_s(rows, sem):
            pltpu.make_async_copy(table_ref.at[pl.ds(0, CH)], rows, sem).wait()

        def wait_d(d_v, sem):
            pltpu.make_async_copy(src_ref.at[pl.ds(0, CH)], d_v, sem).wait()

        dcpy(0, d_a, si_a)
        gath(0, rows_a, sg_a)
        dcpy(1, d_b, si_b)
        gath(1, rows_b, sg_b)

        ntrip = SBLK // 3

        def triple(k, _):
            j0 = 3 * k

            wait_g(rows_a, sg_a)
            wait_d(d_a, si_a)
            scat(rows_a, d_a, ss_a)

            @pl.when(k > 0)
            def _():
                wait_s(rows_c, ss_c)
            dcpy(j0 + 2, d_c, si_c)
            gath(j0 + 2, rows_c, sg_c)

            wait_g(rows_b, sg_b)
            wait_d(d_b, si_b)
            scat(rows_b, d_b, ss_b)

            @pl.when(k < ntrip - 1)
            def _():
                wait_s(rows_a, ss_a)
                dcpy(j0 + 3, d_a, si_a)
                gath(j0 + 3, rows_a, sg_a)

            wait_g(rows_c, sg_c)
            wait_d(d_c, si_c)
            scat(rows_c, d_c, ss_c)

            @pl.when(k < ntrip - 1)
            def _():
                wait_s(rows_b, ss_b)
                dcpy(j0 + 4, d_b, si_b)
                gath(j0 + 4, rows_b, sg_b)
            return 0
        lax.fori_loop(0, ntrip, triple, 0)
        wait_s(rows_a, ss_a)
        wait_s(rows_b, ss_b)
        wait_s(rows_c, ss_c)
        return 0
    lax.fori_loop(0, nsb, superblock, 0)

    plsc.subcore_barrier()

    pltpu.sync_copy(acc.at[pl.ds(base, rpt)], out_s_ref.at[pl.ds(c * n_pad + base, rpt)])


def _segment_sum(table2, src1, dst1, n_pad, e):
    mesh = plsc.VectorSubcoreMesh(core_axis_name="c", subcore_axis_name="s",
                                  num_cores=NC, num_subcores=NS)
    return pl.kernel(
        functools.partial(_sum_body, n_pad, e),
        out_type=jax.ShapeDtypeStruct((NC * n_pad, DH), jnp.float32),
        mesh=mesh,
        scratch_types=[
            pltpu.VMEM((SBLK * CH,), jnp.int32),
            pltpu.VMEM((CH,), jnp.int32),
            pltpu.VMEM((CH,), jnp.int32),
            pltpu.VMEM((CH,), jnp.int32),
            pltpu.VMEM((CH, DH), jnp.float32),
            pltpu.VMEM((CH, DH), jnp.float32),
            pltpu.VMEM((CH, DH), jnp.float32),
            pltpu.VMEM_SHARED((n_pad, DH), jnp.float32),
            pltpu.SemaphoreType.DMA,
            pltpu.SemaphoreType.DMA,
            pltpu.SemaphoreType.DMA,
            pltpu.SemaphoreType.DMA,
            pltpu.SemaphoreType.DMA,
            pltpu.SemaphoreType.DMA,
            pltpu.SemaphoreType.DMA,
            pltpu.SemaphoreType.DMA,
            pltpu.SemaphoreType.DMA,
        ],
        compiler_params=pltpu.CompilerParams(needs_layout_passes=False),
    )(table2, src1, dst1)


def _counts_body(n, e, bn, dst_ref, out_c_ref, dst_v, cnt2):
    c = lax.axis_index("c")
    s = lax.axis_index("s")
    w = s * NC + c
    ept = e // (NC * NS)
    nw = ept // 16
    nb = (n + bn - 1) // bn
    nrow = nb * bn // 128
    dummy = n + 16
    zero16 = jnp.zeros((16,), jnp.float32)
    one16 = jnp.ones((16,), jnp.float32)

    def zf(i, _):
        cnt2[i // 8, pl.ds((i % 8) * 16, 16)] = zero16
        return 0
    lax.fori_loop(0, nrow * 8, zf, 0)

    dst_v[pl.ds(nw * 16, 16)] = jnp.full((16,), dummy, jnp.int32)
    pltpu.sync_copy(dst_ref.at[pl.ds(w * ept, ept)], dst_v.at[pl.ds(0, ept)])

    def add(i, _):
        iv = dst_v[pl.ds(i * 16, 16)]
        plsc.addupdate_scatter(cnt2, [jnp.right_shift(iv, 7),
                                      jnp.bitwise_and(iv, 127)], one16)
        return 0
    lax.fori_loop(0, nw + (1 if ept % 16 else 0), add, 0)

    def wb(i, _):
        pltpu.sync_copy(cnt2.at[pl.ds(8 * i, 8)], out_c_ref.at[i].at[w])
        return 0
    lax.fori_loop(0, nb, wb, 0)


def _counts(dst, n, e, bn):
    mesh = plsc.VectorSubcoreMesh(core_axis_name="c", subcore_axis_name="s",
                                  num_cores=NC, num_subcores=NS)
    ept = e // (NC * NS)
    stage = ((ept + 15) // 16) * 16 + 16
    nb = (n + bn - 1) // bn
    return pl.kernel(
        functools.partial(_counts_body, n, e, bn),
        out_type=jax.ShapeDtypeStruct((nb, NC * NS, 8, 128), jnp.float32),
        mesh=mesh,
        scratch_types=[
            pltpu.VMEM((stage,), jnp.int32),
            pltpu.VMEM((nb * bn // 128, 128), jnp.float32),
        ],
        compiler_params=pltpu.CompilerParams(needs_layout_passes=False),
    )(dst)


def _post_body(hs_ref, s_ref, c_ref, ws_ref, wn_ref, out_ref):
    bn = hs_ref.shape[0]
    nt = NC * NS
    m = c_ref[0].reshape(nt * 8, 128)
    sel = (lax.broadcasted_iota(jnp.int32, (8, nt * 8), 1) % 8
           == lax.broadcasted_iota(jnp.int32, (8, nt * 8), 0)).astype(jnp.float32)
    cnt8 = lax.dot_general(sel, m, (((1,), (0,)), ((), ())),
                           preferred_element_type=jnp.float32)
    rowsel = (lax.broadcasted_iota(jnp.int32, (bn, 8), 1)
              == lax.broadcasted_iota(jnp.int32, (bn, 8), 0) // 128).astype(jnp.float32)
    rep = lax.dot_general(rowsel, cnt8, (((1,), (0,)), ((), ())),
                          preferred_element_type=jnp.float32)
    colmask = (lax.broadcasted_iota(jnp.int32, (bn, 128), 1)
               == lax.broadcasted_iota(jnp.int32, (bn, 128), 0) % 128).astype(jnp.float32)
    cnt = lax.dot_general(rep * colmask, jnp.ones((128, 1), jnp.float32),
                          (((1,), (0,)), ((), ())),
                          preferred_element_type=jnp.float32)
    cnt = jnp.maximum(cnt, 1.0)
    neigh = jnp.concatenate([s_ref[0], s_ref[1]], axis=1) / cnt
    z = lax.dot_general(hs_ref[...], ws_ref[...], (((1,), (1,)), ((), ())),
                        preferred_element_type=jnp.float32)
    z = z + lax.dot_general(neigh, wn_ref[...], (((1,), (1,)), ((), ())),
                            preferred_element_type=jnp.float32)
    z = jnp.maximum(z, 0.0)
    ss = jnp.sum(z * z, axis=1, keepdims=True)
    inv = jnp.where(ss > 0.0, lax.rsqrt(ss), 1.0)
    out_ref[...] = z * inv


def _post(h_self, summed, counts, w_self, w_neigh, bn, n_pad):
    n, d = h_self.shape
    grid = (n + bn - 1) // bn
    summed3 = summed.reshape(NC, n_pad, DH)
    return pl.pallas_call(
        _post_body,
        grid=(grid,),
        in_specs=[
            pl.BlockSpec((bn, d), lambda i: (i, 0)),
            pl.BlockSpec((NC, bn, DH), lambda i: (0, i, 0)),
            pl.BlockSpec((1, NC * NS, 8, 128), lambda i: (i, 0, 0, 0)),
            pl.BlockSpec((d, d), lambda i: (0, 0)),
            pl.BlockSpec((d, d), lambda i: (0, 0)),
        ],
        out_specs=pl.BlockSpec((bn, d), lambda i: (i, 0)),
        out_shape=jax.ShapeDtypeStruct((n, d), jnp.float32),
    )(h_self, summed3, counts, w_self, w_neigh)


def kernel(h_neighbour, h_self, edge_index, W_pre, W_self, W_neigh):
    n = h_self.shape[0]
    e = edge_index.shape[1]
    n_pad = ((n + NS * 8 - 1) // (NS * 8)) * (NS * 8)
    ei = edge_index.astype(jnp.int32)
    src = ei[0]
    dst = ei[1]
    ept = e // NS
    junk = n + 16
    pad_e = NSLOT * CH - ept
    src16 = jnp.concatenate([src.reshape(NS, ept),
                             jnp.zeros((NS, pad_e), jnp.int32)], axis=1)
    src1 = jnp.concatenate([src16[None], src16[None] + n_pad], axis=0).reshape(-1)
    dst1 = jnp.concatenate([dst.reshape(NS, ept),
                            jnp.full((NS, pad_e), junk, jnp.int32)], axis=1).reshape(-1)

    table2 = _pre_matmul(h_neighbour, W_pre, bn=1000, n_pad=n_pad)
    out_s = _segment_sum(table2, src1, dst1, n_pad, e)
    out_c = _counts(dst, n, e, bn=1024)
    return _post(h_self, out_s, out_c, W_self, W_neigh, bn=1024, n_pad=n_pad)


# speedup vs baseline: 1.1001x; 1.1001x over previous
import functools

import jax
import jax.numpy as jnp
from jax import lax
from jax.experimental import pallas as pl
from jax.experimental.pallas import tpu as pltpu
from jax.experimental.pallas import tpu_sc as plsc

NC = 2
NS = 16
DH = 128
CH = 80
NACC = 10240
NSLOT = 128
SBLK = 64


def _pre_body(x_ref, w_ref, out_ref):
    y = lax.dot_general(x_ref[...], w_ref[...], (((1,), (1,)), ((), ())),
                        preferred_element_type=jnp.float32)
    y = jnp.maximum(y, 0.0)
    out_ref[0] = y[:, :DH]
    out_ref[1] = y[:, DH:]


def _pre_matmul(h_neighbour, w_pre, bn, n_pad):
    n, d = h_neighbour.shape
    grid = n // bn
    out = pl.pallas_call(
        _pre_body,
        grid=(grid,),
        in_specs=[
            pl.BlockSpec((bn, d), lambda i: (i, 0)),
            pl.BlockSpec((d, d), lambda i: (0, 0)),
        ],
        out_specs=pl.BlockSpec((NC, bn, DH), lambda i: (0, i, 0)),
        out_shape=jax.ShapeDtypeStruct((NC, n_pad, DH), jnp.float32),
    )(h_neighbour, w_pre)
    return out.reshape(NC * n_pad, DH)


def _sum_body(n_pad, e, table_ref, src_ref, dst_ref, out_s_ref, out_c_ref,
              src_blk, d_a, d_b, rows_a, rows_b, cnt2, acc,
              sg_a, sg_b, ss_a, ss_b, si_a, si_b):
    c = lax.axis_index("c")
    s = lax.axis_index("s")
    rpt = n_pad // NS
    base = s * rpt
    spt = NSLOT * CH

    w = s * NC + c
    zero16 = jnp.zeros((16,), jnp.float32)
    one16 = jnp.ones((16,), jnp.float32)

    def zero_cnt(i, _):
        cnt2[i // 8, pl.ds((i % 8) * 16, 16)] = zero16
        return 0
    lax.fori_loop(0, (NACC // 128) * 8, zero_cnt, 0)

    def count_chunk(d_v):
        def cadd(t, _):
            iv = d_v[pl.ds(t * 16, 16)]
            plsc.addupdate_scatter(cnt2, [jnp.right_shift(iv, 7),
                                          jnp.bitwise_and(iv, 127)], one16)
            return 0
        lax.fori_loop(0, CH // 16, cadd, 0)

    def zero_rows(i, _):
        rows_a[i // (DH // 16), pl.ds((i % (DH // 16)) * 16, 16)] = zero16
        return 0
    lax.fori_loop(0, CH * (DH // 16), zero_rows, 0)

    nzc = rpt // CH
    rem = rpt % CH

    def zero_acc(i, _):
        pltpu.sync_copy(rows_a, acc.at[pl.ds(base + i * CH, CH)])
        return 0
    lax.fori_loop(0, nzc, zero_acc, 0)
    if rem:
        pltpu.sync_copy(rows_a.at[pl.ds(0, rem)], acc.at[pl.ds(base + nzc * CH, rem)])

    plsc.subcore_barrier()

    sbase = (c * NS + s) * spt
    dbase = s * spt

    nsb = NSLOT // SBLK
    npair = SBLK // 2

    def superblock(sb, _):
        soff = sb * SBLK * CH
        pltpu.sync_copy(src_ref.at[pl.ds(sbase + soff, SBLK * CH)], src_blk)

        def gath(j, rows, sem):
            pltpu.async_copy(table_ref.at[src_blk.at[pl.ds(j * CH, CH)]],
                             rows, sem)

        def dcpy(j, d_v, sem):
            pltpu.async_copy(dst_ref.at[pl.ds(dbase + soff + j * CH, CH)],
                             d_v, sem)

        def scat(rows, d_v, sem):
            pltpu.async_copy(rows, acc.at[d_v], sem, add=True)

        def wait_g(rows, sem):
            pltpu.make_async_copy(table_ref.at[pl.ds(0, CH)], rows, sem).wait()

        def wait_s(rows, sem):
            pltpu.make_async_copy(table_ref.at[pl.ds(0, CH)], rows, sem).wait()

        def wait_d(d_v, sem):
            pltpu.make_async_copy(src_ref.at[pl.ds(0, CH)], d_v, sem).wait()

        dcpy(0, d_a, si_a)
        gath(0, rows_a, sg_a)

        def pair(k, _):
            a = 2 * k
            b = 2 * k + 1
            wait_g(rows_a, sg_a)
            wait_d(d_a, si_a)
            scat(rows_a, d_a, ss_a)
            count_chunk(d_a)

            @pl.when(k > 0)
            def _():
                wait_s(rows_b, ss_b)
            dcpy(b, d_b, si_b)
            gath(b, rows_b, sg_b)

            wait_g(rows_b, sg_b)
            wait_d(d_b, si_b)
            scat(rows_b, d_b, ss_b)
            count_chunk(d_b)

            wait_s(rows_a, ss_a)

            @pl.when(k < npair - 1)
            def _():
                dcpy(a + 2, d_a, si_a)
                gath(a + 2, rows_a, sg_a)
            return 0
        lax.fori_loop(0, npair, pair, 0)
        wait_s(rows_b, ss_b)
        return 0
    lax.fori_loop(0, nsb, superblock, 0)

    plsc.subcore_barrier()

    pltpu.sync_copy(acc.at[pl.ds(base, rpt)], out_s_ref.at[pl.ds(c * n_pad + base, rpt)])

    def wb(i, _):
        pltpu.sync_copy(cnt2.at[pl.ds(8 * i, 8)], out_c_ref.at[i].at[w])
        return 0
    lax.fori_loop(0, NACC // 1024, wb, 0)


def _segment_sum(table2, src1, dst1, n_pad, e):
    mesh = plsc.VectorSubcoreMesh(core_axis_name="c", subcore_axis_name="s",
                                  num_cores=NC, num_subcores=NS)
    return pl.kernel(
        functools.partial(_sum_body, n_pad, e),
        out_type=[jax.ShapeDtypeStruct((NC * n_pad, DH), jnp.float32),
                  jax.ShapeDtypeStruct((NACC // 1024, NC * NS, 8, 128), jnp.float32)],
        mesh=mesh,
        scratch_types=[
            pltpu.VMEM((SBLK * CH,), jnp.int32),
            pltpu.VMEM((CH,), jnp.int32),
            pltpu.VMEM((CH,), jnp.int32),
            pltpu.VMEM((CH, DH), jnp.float32),
            pltpu.VMEM((CH, DH), jnp.float32),
            pltpu.VMEM((NACC // 128, 128), jnp.float32),
            pltpu.VMEM_SHARED((n_pad, DH), jnp.float32),
            pltpu.SemaphoreType.DMA,
            pltpu.SemaphoreType.DMA,
            pltpu.SemaphoreType.DMA,
            pltpu.SemaphoreType.DMA,
            pltpu.SemaphoreType.DMA,
            pltpu.SemaphoreType.DMA,
        ],
        compiler_params=pltpu.CompilerParams(needs_layout_passes=False),
    )(table2, src1, dst1)


def _self_body(x_ref, w_ref, out_ref):
    out_ref[...] = lax.dot_general(x_ref[...], w_ref[...], (((1,), (1,)), ((), ())),
                                   preferred_element_type=jnp.float32)


def _self_matmul(h_self, w_self, bn):
    n, d = h_self.shape
    return pl.pallas_call(
        _self_body,
        grid=(n // bn,),
        in_specs=[pl.BlockSpec((bn, d), lambda i: (i, 0)),
                  pl.BlockSpec((d, d), lambda i: (0, 0))],
        out_specs=pl.BlockSpec((bn, d), lambda i: (i, 0)),
        out_shape=jax.ShapeDtypeStruct((n, d), jnp.float32),
    )(h_self, w_self)


def _post_body(zs_ref, s_ref, c_ref, wn_ref, out_ref):
    bn = zs_ref.shape[0]
    nt = NC * NS
    m = c_ref[0].reshape(nt * 8, 128)
    sel = (lax.broadcasted_iota(jnp.int32, (8, nt * 8), 1) % 8
           == lax.broadcasted_iota(jnp.int32, (8, nt * 8), 0)).astype(jnp.float32)
    cnt8 = lax.dot_general(sel, m, (((1,), (0,)), ((), ())),
                           preferred_element_type=jnp.float32)
    rowsel = (lax.broadcasted_iota(jnp.int32, (bn, 8), 1)
              == lax.broadcasted_iota(jnp.int32, (bn, 8), 0) // 128).astype(jnp.float32)
    rep = lax.dot_general(rowsel, cnt8, (((1,), (0,)), ((), ())),
                          preferred_element_type=jnp.float32)
    colmask = (lax.broadcasted_iota(jnp.int32, (bn, 128), 1)
               == lax.broadcasted_iota(jnp.int32, (bn, 128), 0) % 128).astype(jnp.float32)
    cnt = lax.dot_general(rep * colmask, jnp.ones((128, 1), jnp.float32),
                          (((1,), (0,)), ((), ())),
                          preferred_element_type=jnp.float32)
    cnt = jnp.maximum(cnt * 0.5, 1.0)
    neigh = jnp.concatenate([s_ref[0], s_ref[1]], axis=1) / cnt
    z = zs_ref[...] + lax.dot_general(neigh, wn_ref[...], (((1,), (1,)), ((), ())),
                                      preferred_element_type=jnp.float32)
    z = jnp.maximum(z, 0.0)
    ss = jnp.sum(z * z, axis=1, keepdims=True)
    inv = jnp.where(ss > 0.0, lax.rsqrt(ss), 1.0)
    out_ref[...] = z * inv


def _post(zs, summed, counts, w_neigh, bn, n_pad):
    n, d = zs.shape
    grid = (n + bn - 1) // bn
    summed3 = summed.reshape(NC, n_pad, DH)
    return pl.pallas_call(
        _post_body,
        grid=(grid,),
        in_specs=[
            pl.BlockSpec((bn, d), lambda i: (i, 0)),
            pl.BlockSpec((NC, bn, DH), lambda i: (0, i, 0)),
            pl.BlockSpec((1, NC * NS, 8, 128), lambda i: (i, 0, 0, 0)),
            pl.BlockSpec((d, d), lambda i: (0, 0)),
        ],
        out_specs=pl.BlockSpec((bn, d), lambda i: (i, 0)),
        out_shape=jax.ShapeDtypeStruct((n, d), jnp.float32),
    )(zs, summed3, counts, w_neigh)


def kernel(h_neighbour, h_self, edge_index, W_pre, W_self, W_neigh):
    n = h_self.shape[0]
    e = edge_index.shape[1]
    n_pad = ((n + NS * 8 - 1) // (NS * 8)) * (NS * 8)
    ei = edge_index.astype(jnp.int32)
    src = ei[0]
    dst = ei[1]
    ept = e // NS
    junk = n + 16
    pad_e = NSLOT * CH - ept
    src16 = jnp.concatenate([src.reshape(NS, ept),
                             jnp.zeros((NS, pad_e), jnp.int32)], axis=1)
    src1 = jnp.concatenate([src16[None], src16[None] + n_pad], axis=0).reshape(-1)
    dst1 = jnp.concatenate([dst.reshape(NS, ept),
                            jnp.full((NS, pad_e), junk, jnp.int32)], axis=1).reshape(-1)

    table2 = _pre_matmul(h_neighbour, W_pre, bn=1000, n_pad=n_pad)
    out_s, out_c = _segment_sum(table2, src1, dst1, n_pad, e)
    zs = _self_matmul(h_self, W_self, bn=1000)
    return _post(zs, out_s, out_c, W_neigh, bn=1024, n_pad=n_pad)
